# baseline (speedup 1.0000x reference)
import jax
import jax.numpy as jnp
from jax.experimental import pallas as pl


def _add_kernel(a_ref, b_ref, o_ref):
    o_ref[...] = a_ref[...] + b_ref[...]


def _gcn_conv(x, edge_index, edge_weight, W, b):
    N = x.shape[0]
    row = edge_index[0]
    col = edge_index[1]
    loop = jnp.arange(N, dtype=row.dtype)
    row_f = jnp.concatenate([row, loop])
    col_f = jnp.concatenate([col, loop])
    ew_f = jnp.concatenate([edge_weight, jnp.ones((N,), dtype=x.dtype)])
    deg = jnp.zeros((N,), dtype=x.dtype).at[col_f].add(ew_f)
    deg_inv_sqrt = jnp.where(deg > 0, deg ** -0.5, 0.0)
    norm = deg_inv_sqrt[row_f] * ew_f * deg_inv_sqrt[col_f]
    h = x @ W
    msg = h[row_f] * norm[:, None]
    out = jnp.zeros((N, W.shape[1]), dtype=x.dtype).at[col_f].add(msg)
    return out + b


def kernel(x, edge_index, edge_type, edge_weight, W_pos, b_pos, W_neg, b_neg):
    pos_w = jnp.where(edge_type == 0, edge_weight, 0.0)
    neg_w = jnp.where(edge_type == 1, edge_weight, 0.0)
    pos_out = _gcn_conv(x, edge_index, pos_w, W_pos, b_pos)
    neg_out = _gcn_conv(x, edge_index, neg_w, W_neg, b_neg)
    return pl.pallas_call(
        _add_kernel,
        out_shape=jax.ShapeDtypeStruct(pos_out.shape, pos_out.dtype),
    )(pos_out, neg_out)


# speedup vs baseline: 17.4974x; 17.4974x over previous
import functools

import jax
import jax.numpy as jnp
from jax import lax
from jax.experimental import pallas as pl
from jax.experimental.pallas import tpu as pltpu
from jax.experimental.pallas import tpu_sc as plsc

N = 10000
D = 128
E = 320000
NC = 2
NS = 16
L = 16
NT = NC * NS
N2P = 20480
EP = 327680
EPT = EP // NT
CH = 1024
NCHUNK = EPT // CH
DCH = 2048
NDCH = EPT // DCH
NACC = 10240


def _matmul_body(x_ref, w_ref, o_ref):
    o_ref[0] = jnp.dot(x_ref[...], w_ref[0], preferred_element_type=jnp.float32)


def _compute_h2(x, w_st):
    bn = 1000
    return pl.pallas_call(
        _matmul_body,
        grid=(2, N // bn),
        in_specs=[
            pl.BlockSpec((bn, D), lambda i, j: (j, 0)),
            pl.BlockSpec((1, D, D), lambda i, j: (i, 0, 0)),
        ],
        out_specs=pl.BlockSpec((1, bn, D), lambda i, j: (i, j, 0)),
        out_shape=jax.ShapeDtypeStruct((2, N, D), jnp.float32),
    )(x, w_st)


def _deg_body(idst2_h, ew_h, degp_h, dgi, dgw, zb, deg_sh, sem):
    c = lax.axis_index("c")
    s = lax.axis_index("s")

    def _zb(i, _):
        zb[pl.ds(i * L, L)] = jnp.zeros((L,), jnp.float32)
        return 0
    lax.fori_loop(0, (N2P // NS) // L, _zb, 0)
    rbase = pl.multiple_of(s * (N2P // NS), 128)
    pltpu.sync_copy(zb, deg_sh.at[pl.ds(rbase, N2P // NS)])
    plsc.subcore_barrier()

    ebase = (c * NS + s) * EPT
    for dc in range(NDCH):
        r0 = pl.multiple_of((ebase + dc * DCH) // 128, 16)
        pltpu.sync_copy(idst2_h.at[pl.ds(r0, 16)], dgi)
        pltpu.sync_copy(ew_h.at[pl.ds(r0, 16)], dgw)
        descs = [pltpu.async_copy(dgw.at[j], deg_sh.at[dgi.at[j]], sem,
                                  add=True) for j in range(16)]
        for dsc in descs:
            dsc.wait()
    plsc.subcore_barrier()
    off = pl.multiple_of(c * N2P + rbase, 128)
    pltpu.sync_copy(deg_sh.at[pl.ds(rbase, N2P // NS)],
                    degp_h.at[pl.ds(off, N2P // NS)])


_deg_kernel = functools.partial(
    pl.kernel,
    out_type=jax.ShapeDtypeStruct((2 * N2P,), jnp.float32),
    mesh=plsc.VectorSubcoreMesh(core_axis_name="c", subcore_axis_name="s"),
    compiler_params=pltpu.CompilerParams(needs_layout_passes=False),
    scratch_types=[
        pltpu.VMEM((16, 128), jnp.int32),
        pltpu.VMEM((16, 128), jnp.float32),
        pltpu.VMEM((N2P // NS,), jnp.float32),
        pltpu.VMEM_SHARED((N2P,), jnp.float32),
        pltpu.SemaphoreType.DMA,
    ],
)(_deg_body)


def _rsqrt_body(p_ref, dis_ref, inv_ref):
    deg = p_ref[0] + p_ref[1] + 1.0
    dis_ref[...] = lax.rsqrt(deg)
    inv_ref[...] = 1.0 / deg


def _compute_dis(deg_part):
    return pl.pallas_call(
        _rsqrt_body,
        out_shape=(
            jax.ShapeDtypeStruct((N2P // 128, 128), jnp.float32),
            jax.ShapeDtypeStruct((N2P // 128, 128), jnp.float32),
        ),
    )(deg_part.reshape(2, N2P // 128, 128))


_GDN = lax.GatherDimensionNumbers(
    offset_dims=(), collapsed_slice_dims=(0,), start_index_map=(0,))


def _lane_bcast(v, l):
    return lax.gather(v, jnp.full((L, 1), l, jnp.int32), _GDN, (1,),
                      mode=lax.GatherScatterMode.PROMISE_IN_BOUNDS)


def _sweep_body(h2, dis_h, isrc_h, idst_h, idst2_h, ew_h, part_h,
                dis2_loc, rows, fb, bb, isrc, idst, idst2, ewb, coefb,
                part_sh, sem):
    c = lax.axis_index("c")
    s = lax.axis_index("s")

    def _zrow(g, _):
        for v in range(8):
            rows[g, pl.ds(v * L, L)] = jnp.zeros((L,), jnp.float32)
        return 0
    lax.fori_loop(0, 128, _zrow, 0)

    def _init(chunks):
        ibase = pl.multiple_of(s * 640, 8)
        for zoff, zn in chunks:
            pltpu.sync_copy(rows.at[pl.ds(0, zn)],
                            part_sh.at[pl.ds(ibase + zoff, zn)])

    @pl.when(s < 15)
    def _():
        _init(((0, 128), (128, 128), (256, 128), (384, 128), (512, 128)))

    @pl.when(s == 15)
    def _():
        _init(((0, 128), (128, 128), (256, 144)))

    pltpu.sync_copy(dis_h, dis2_loc)
    plsc.subcore_barrier()

    mbase = (c * NS + s) * EPT

    def _chunk(k, _):
        r0 = pl.multiple_of((mbase + k * CH) // 128, 8)
        pltpu.sync_copy(isrc_h.at[pl.ds(r0, 8)], isrc)
        pltpu.sync_copy(idst_h.at[pl.ds(r0, 8)], idst)
        pltpu.sync_copy(idst2_h.at[pl.ds(r0, 8)], idst2)
        pltpu.sync_copy(ew_h.at[pl.ds(r0, 8)], ewb)
        for q in range(8):
            gather = pltpu.async_copy(h2.at[isrc.at[q]],
                                      rows.at[pl.ds(0, 128)], sem)

            def _cf(g, _, q=q):
                oo = g * L
                sv = isrc[q, pl.ds(oo, L)]
                dv = idst2[q, pl.ds(oo, L)]
                a = plsc.load_gather(dis2_loc, [sv])
                b = plsc.load_gather(dis2_loc, [dv])
                w = ewb[q, pl.ds(oo, L)]
                coefb[pl.ds(oo, L)] = a * b * w
                return 0
            lax.fori_loop(0, 8, _cf, 0)

            gather.wait()

            def _sg(g, _):
                cv = coefb[pl.ds(g * L, L)]
                for l in range(L):
                    cl = _lane_bcast(cv, l)
                    e = g * L + l
                    for v in range(8):
                        rows[e, pl.ds(v * L, L)] = (
                            rows[e, pl.ds(v * L, L)] * cl)
                return 0
            lax.fori_loop(0, 8, _sg, 0)

            pltpu.sync_copy(rows.at[pl.ds(0, 128)],
                            part_sh.at[idst.at[q]], add=True)
        return 0
    lax.fori_loop(0, NCHUNK, _chunk, 0)

    plsc.subcore_barrier()

    def _export(chunks):
        ebase = pl.multiple_of(s * 640, 8)
        for hc, nr in chunks:
            roff = pl.multiple_of(ebase + hc * 32, 8)
            pltpu.sync_copy(part_sh.at[pl.ds(roff, nr)], fb.at[pl.ds(0, nr)])

            def _pk(r, _):
                for v in range(4):
                    a = fb[r, pl.ds(32 * v, L)]
                    b = fb[r, pl.ds(32 * v + L, L)]
                    bb[r, pl.ds(32 * v, 2 * L)] = plsc.pack(
                        a, b, format=plsc.PackFormat.INTERLEAVED)
                return 0
            lax.fori_loop(0, nr, _pk, 0)
            off = pl.multiple_of(c * N + roff, 8)
            pltpu.sync_copy(bb.at[pl.ds(0, nr)], part_h.at[pl.ds(off, nr)])

    @pl.when(s < 15)
    def _():
        _export(tuple((hc, 32) for hc in range(20)))

    @pl.when(s == 15)
    def _():
        _export(tuple((hc, 32) for hc in range(12)) + ((12, 16),))


_sweep_kernel = functools.partial(
    pl.kernel,
    out_type=jax.ShapeDtypeStruct((2 * N, D), jnp.bfloat16),
    mesh=plsc.VectorSubcoreMesh(core_axis_name="c", subcore_axis_name="s"),
    compiler_params=pltpu.CompilerParams(needs_layout_passes=False),
    scratch_types=[
        pltpu.VMEM((N2P,), jnp.float32),
        pltpu.VMEM((128, D), jnp.float32),
        pltpu.VMEM((32, D), jnp.float32),
        pltpu.VMEM((32, D), jnp.bfloat16),
        pltpu.VMEM((8, 128), jnp.int32),
        pltpu.VMEM((8, 128), jnp.int32),
        pltpu.VMEM((8, 128), jnp.int32),
        pltpu.VMEM((8, 128), jnp.float32),
        pltpu.VMEM((128,), jnp.float32),
        pltpu.VMEM_SHARED((N, D), jnp.float32),
        pltpu.SemaphoreType.DMA,
    ],
)(_sweep_body)


def _comb_body(p0, p1, h0, h1, i0, i1, b, o):
    o[...] = (p0[0] + p1[0]
              + h0[0] * i0[0]
              + h1[0] * i1[0]
              + b[0])


def _combine(part, h2, inv2, b_sum):
    bn = 1000
    nb = N // bn
    return pl.pallas_call(
        _comb_body,
        grid=(nb,),
        in_specs=[
            pl.BlockSpec((1, bn, D), lambda j: (0, j, 0)),
            pl.BlockSpec((1, bn, D), lambda j: (1, j, 0)),
            pl.BlockSpec((1, bn, D), lambda j: (0, j, 0)),
            pl.BlockSpec((1, bn, D), lambda j: (1, j, 0)),
            pl.BlockSpec((1, bn, 1), lambda j: (0, j, 0)),
            pl.BlockSpec((1, bn, 1), lambda j: (1, j, 0)),
            pl.BlockSpec((1, 1, D), lambda j: (0, 0, 0)),
        ],
        out_specs=pl.BlockSpec((bn, D), lambda j: (j, 0)),
        out_shape=jax.ShapeDtypeStruct((N, D), jnp.float32),
    )(part, part, h2, h2, inv2, inv2, b_sum.reshape(1, 1, D))


def kernel(x, edge_index, edge_type, edge_weight, W_pos, b_pos, W_neg, b_neg):
    ei = edge_index.astype(jnp.int32)
    t = edge_type.astype(jnp.int32)
    row, col = ei[0], ei[1]
    pad = EP - E
    src2 = jnp.pad(row + t * N, (0, pad))
    dst = jnp.pad(col, (0, pad))
    dst2 = jnp.pad(col + t * N, (0, pad))
    ew = jnp.pad(edge_weight.astype(jnp.float32), (0, pad))
    idst2_h = dst2.reshape(-1, 128)
    ew_h = ew.reshape(-1, 128)

    h2 = _compute_h2(x, jnp.stack([W_pos, W_neg]))
    deg_part = _deg_kernel(idst2_h, ew_h)
    dis2, inv2 = _compute_dis(deg_part)
    pb = _sweep_kernel(h2.reshape(2 * N, D), dis2.reshape(N2P),
                       src2.reshape(-1, 128), dst.reshape(-1, 128),
                       idst2_h, ew_h)
    part = (pb.reshape(2 * N, 4, 16, 2).transpose(0, 1, 3, 2)
            .reshape(2, N, D).astype(jnp.float32))
    inv3 = inv2.reshape(N2P)[:2 * N].reshape(2, N, 1)
    return _combine(part, h2, inv3, b_pos + b_neg)
